# baseline (speedup 1.0000x reference)
import functools

import jax
import jax.numpy as jnp
from jax.experimental import pallas as pl
from jax.experimental.pallas import tpu as pltpu


def _body(x_ref, phi_ref, ehi_ref, ctab_ref, out_ref):
    nb = x_ref.shape[0]
    zrow = jnp.zeros((1, x_ref.shape[2]), dtype=jnp.bfloat16)
    for i in range(nb):
        vp = x_ref[i, 0:1, :]
        ve = x_ref[i, 1:2, :]
        g_p = (phi_ref[:, :] >= vp).astype(jnp.bfloat16)
        g_e = (ehi_ref[:, :] >= ve).astype(jnp.bfloat16)
        oh_p = g_p - jnp.concatenate([zrow, g_p[:-1, :]], axis=0)
        oh_e = g_e - jnp.concatenate([zrow, g_e[:-1, :]], axis=0)
        oh = jnp.concatenate([oh_p, oh_e], axis=0)
        out_ref[i] = jnp.dot(ctab_ref[:, :], oh,
                             preferred_element_type=jnp.float32)


@functools.partial(jax.jit, static_argnames=("interpret",))
def kernel(x, pitch_bins, energy_bins, pitch_embedding, energy_embedding,
           interpret=False):
    B, _, T = x.shape
    N, H = pitch_embedding.shape
    Tt = 1024
    Bb = 8

    inf = jnp.array([jnp.inf], dtype=jnp.float32)
    p_hi = jnp.concatenate([pitch_bins, inf])[:, None]
    e_hi = jnp.concatenate([energy_bins, inf])[:, None]
    ctab = jnp.concatenate(
        [pitch_embedding.T, energy_embedding.T], axis=1,
    ).astype(jnp.bfloat16)

    grid = (B // Bb, T // Tt)
    return pl.pallas_call(
        _body,
        grid=grid,
        in_specs=[
            pl.BlockSpec((Bb, 2, Tt), lambda b, j: (b, 0, j)),
            pl.BlockSpec((N, 1), lambda b, j: (0, 0)),
            pl.BlockSpec((N, 1), lambda b, j: (0, 0)),
            pl.BlockSpec((H, 2 * N), lambda b, j: (0, 0)),
        ],
        out_specs=pl.BlockSpec((Bb, H, Tt), lambda b, j: (b, 0, j)),
        out_shape=jax.ShapeDtypeStruct((B, H, T), jnp.float32),
        compiler_params=pltpu.CompilerParams(
            dimension_semantics=("parallel", "parallel")),
        interpret=interpret,
    )(x, p_hi, e_hi, ctab)


# speedup vs baseline: 1.0084x; 1.0084x over previous
import functools

import jax
import jax.numpy as jnp
from jax.experimental import pallas as pl
from jax.experimental.pallas import tpu as pltpu


def _body(x_ref, phi_ref, ehi_ref, ctab_ref, out_ref):
    nb = x_ref.shape[0]
    zrow = jnp.zeros((1, x_ref.shape[2]), dtype=jnp.bfloat16)
    for i in range(nb):
        vp = x_ref[i, 0:1, :]
        ve = x_ref[i, 1:2, :]
        g_p = (phi_ref[:, :] >= vp).astype(jnp.bfloat16)
        g_e = (ehi_ref[:, :] >= ve).astype(jnp.bfloat16)
        oh_p = g_p - jnp.concatenate([zrow, g_p[:-1, :]], axis=0)
        oh_e = g_e - jnp.concatenate([zrow, g_e[:-1, :]], axis=0)
        oh = jnp.concatenate([oh_p, oh_e], axis=0)
        out_ref[i] = jax.lax.dot_general(
            ctab_ref[:, :], oh, (((0,), (0,)), ((), ())),
            preferred_element_type=jnp.float32)


@functools.partial(jax.jit, static_argnames=("interpret",))
def kernel(x, pitch_bins, energy_bins, pitch_embedding, energy_embedding,
           interpret=False):
    B, _, T = x.shape
    N, H = pitch_embedding.shape
    Tt = 2048
    Bb = 8

    inf = jnp.array([jnp.inf], dtype=jnp.float32)
    p_hi = jnp.concatenate([pitch_bins, inf])[:, None]
    e_hi = jnp.concatenate([energy_bins, inf])[:, None]
    ctab = jnp.concatenate(
        [pitch_embedding, energy_embedding], axis=0,
    ).astype(jnp.bfloat16)

    grid = (B // Bb, T // Tt)
    return pl.pallas_call(
        _body,
        grid=grid,
        in_specs=[
            pl.BlockSpec((Bb, 2, Tt), lambda b, j: (b, 0, j)),
            pl.BlockSpec((N, 1), lambda b, j: (0, 0)),
            pl.BlockSpec((N, 1), lambda b, j: (0, 0)),
            pl.BlockSpec((2 * N, H), lambda b, j: (0, 0)),
        ],
        out_specs=pl.BlockSpec((Bb, H, Tt), lambda b, j: (b, 0, j)),
        out_shape=jax.ShapeDtypeStruct((B, H, T), jnp.float32),
        compiler_params=pltpu.CompilerParams(
            dimension_semantics=("parallel", "parallel")),
        interpret=interpret,
    )(x, p_hi, e_hi, ctab)


# speedup vs baseline: 1.0104x; 1.0020x over previous
import functools

import jax
import jax.numpy as jnp
from jax.experimental import pallas as pl
from jax.experimental.pallas import tpu as pltpu


def _body(x_ref, phi_ref, ehi_ref, ptab_ref, etab_ref, out_ref):
    nb = x_ref.shape[0]
    zrow = jnp.zeros((1, x_ref.shape[2]), dtype=jnp.bfloat16)
    for i in range(nb):
        vp = x_ref[i, 0:1, :]
        ve = x_ref[i, 1:2, :]
        g_p = (phi_ref[:, :] >= vp).astype(jnp.bfloat16)
        g_e = (ehi_ref[:, :] >= ve).astype(jnp.bfloat16)
        oh_p = g_p - jnp.concatenate([zrow, g_p[:-1, :]], axis=0)
        oh_e = g_e - jnp.concatenate([zrow, g_e[:-1, :]], axis=0)
        dn = (((0,), (0,)), ((), ()))
        out_ref[i] = (
            jax.lax.dot_general(ptab_ref[:, :], oh_p, dn,
                                preferred_element_type=jnp.float32)
            + jax.lax.dot_general(etab_ref[:, :], oh_e, dn,
                                  preferred_element_type=jnp.float32))


@functools.partial(jax.jit, static_argnames=("interpret",))
def kernel(x, pitch_bins, energy_bins, pitch_embedding, energy_embedding,
           interpret=False):
    B, _, T = x.shape
    N, H = pitch_embedding.shape
    Tt = 2048
    Bb = 8

    inf = jnp.array([jnp.inf], dtype=jnp.float32)
    p_hi = jnp.concatenate([pitch_bins, inf])[:, None]
    e_hi = jnp.concatenate([energy_bins, inf])[:, None]
    ptab = pitch_embedding.astype(jnp.bfloat16)
    etab = energy_embedding.astype(jnp.bfloat16)

    grid = (B // Bb, T // Tt)
    return pl.pallas_call(
        _body,
        grid=grid,
        in_specs=[
            pl.BlockSpec((Bb, 2, Tt), lambda b, j: (b, 0, j)),
            pl.BlockSpec((N, 1), lambda b, j: (0, 0)),
            pl.BlockSpec((N, 1), lambda b, j: (0, 0)),
            pl.BlockSpec((N, H), lambda b, j: (0, 0)),
            pl.BlockSpec((N, H), lambda b, j: (0, 0)),
        ],
        out_specs=pl.BlockSpec((Bb, H, Tt), lambda b, j: (b, 0, j)),
        out_shape=jax.ShapeDtypeStruct((B, H, T), jnp.float32),
        compiler_params=pltpu.CompilerParams(
            dimension_semantics=("parallel", "parallel")),
        interpret=interpret,
    )(x, p_hi, e_hi, ptab, etab)


# speedup vs baseline: 1.0210x; 1.0105x over previous
import functools

import jax
import jax.numpy as jnp
from jax.experimental import pallas as pl
from jax.experimental.pallas import tpu as pltpu


def _body(x_ref, hi_ref, ctab_ref, out_ref):
    nb = x_ref.shape[0]
    zrow = jnp.zeros((1, x_ref.shape[2]), dtype=jnp.bfloat16)
    for i in range(nb):
        vp = x_ref[i, 0:1, :]
        ve = x_ref[i, 1:2, :]
        n = hi_ref.shape[0] // 2
        g_p = (hi_ref[:n, :] >= vp).astype(jnp.bfloat16)
        g_e = (hi_ref[n:, :] >= ve).astype(jnp.bfloat16)
        oh_p = g_p - jnp.concatenate([zrow, g_p[:-1, :]], axis=0)
        oh_e = g_e - jnp.concatenate([zrow, g_e[:-1, :]], axis=0)
        oh = jnp.concatenate([oh_p, oh_e], axis=0)
        out_ref[i] = jax.lax.dot_general(
            ctab_ref[:, :], oh, (((0,), (0,)), ((), ())),
            preferred_element_type=jnp.float32)


@functools.partial(jax.jit, static_argnames=("interpret",))
def kernel(x, pitch_bins, energy_bins, pitch_embedding, energy_embedding,
           interpret=False):
    B, _, T = x.shape
    N, H = pitch_embedding.shape
    Tt = 2048
    Bb = 8

    inf = jnp.array([jnp.inf], dtype=jnp.float32)
    hi = jnp.concatenate([pitch_bins, inf, energy_bins, inf])[:, None]
    ctab = jnp.concatenate(
        [pitch_embedding, energy_embedding], axis=0,
    ).astype(jnp.bfloat16)

    grid = (B // Bb, T // Tt)
    return pl.pallas_call(
        _body,
        grid=grid,
        in_specs=[
            pl.BlockSpec((Bb, 2, Tt), lambda b, j: (b, 0, j)),
            pl.BlockSpec((2 * N, 1), lambda b, j: (0, 0)),
            pl.BlockSpec((2 * N, H), lambda b, j: (0, 0)),
        ],
        out_specs=pl.BlockSpec((Bb, H, Tt), lambda b, j: (b, 0, j)),
        out_shape=jax.ShapeDtypeStruct((B, H, T), jnp.float32),
        compiler_params=pltpu.CompilerParams(
            dimension_semantics=("parallel", "parallel")),
        interpret=interpret,
    )(x, hi, ctab)


# speedup vs baseline: 1.0269x; 1.0058x over previous
import functools

import jax
import jax.numpy as jnp
from jax.experimental import pallas as pl
from jax.experimental.pallas import tpu as pltpu


def _body(x_ref, hi_ref, ctab_ref, out_ref):
    nb = x_ref.shape[0]
    zrow = jnp.zeros((1, x_ref.shape[2]), dtype=jnp.bfloat16)
    for i in range(nb):
        vp = x_ref[i, 0:1, :]
        ve = x_ref[i, 1:2, :]
        n = hi_ref.shape[0] // 4
        g_p = (hi_ref[:n, :] >= vp).astype(jnp.bfloat16)
        g_e = (hi_ref[n:2 * n, :] >= ve).astype(jnp.bfloat16)
        s_p = (hi_ref[2 * n:3 * n, :] >= vp).astype(jnp.bfloat16)
        s_e = (hi_ref[3 * n:, :] >= ve).astype(jnp.bfloat16)
        oh_p = g_p - s_p
        oh_e = g_e - s_e
        oh = jnp.concatenate([oh_p, oh_e], axis=0)
        out_ref[i] = jax.lax.dot_general(
            ctab_ref[:, :], oh, (((0,), (0,)), ((), ())),
            preferred_element_type=jnp.float32)


@functools.partial(jax.jit, static_argnames=("interpret",))
def kernel(x, pitch_bins, energy_bins, pitch_embedding, energy_embedding,
           interpret=False):
    B, _, T = x.shape
    N, H = pitch_embedding.shape
    Tt = 2048
    Bb = 8

    inf = jnp.array([jnp.inf], dtype=jnp.float32)
    ninf = jnp.array([-jnp.inf], dtype=jnp.float32)
    hi = jnp.concatenate([pitch_bins, inf, energy_bins, inf,
                          ninf, pitch_bins[:-1], inf,
                          ninf, energy_bins[:-1], inf])[:, None]
    ctab = jnp.concatenate(
        [pitch_embedding, energy_embedding], axis=0,
    ).astype(jnp.bfloat16)

    grid = (B // Bb, T // Tt)
    return pl.pallas_call(
        _body,
        grid=grid,
        in_specs=[
            pl.BlockSpec((Bb, 2, Tt), lambda b, j: (b, 0, j)),
            pl.BlockSpec((4 * N, 1), lambda b, j: (0, 0)),
            pl.BlockSpec((2 * N, H), lambda b, j: (0, 0)),
        ],
        out_specs=pl.BlockSpec((Bb, H, Tt), lambda b, j: (b, 0, j)),
        out_shape=jax.ShapeDtypeStruct((B, H, T), jnp.float32),
        compiler_params=pltpu.CompilerParams(
            dimension_semantics=("parallel", "parallel")),
        interpret=interpret,
    )(x, hi, ctab)
